# baseline (speedup 1.0000x reference)
import jax
import jax.numpy as jnp
from jax import lax
from jax.experimental import pallas as pl
from jax.experimental.pallas import tpu as pltpu
from jax.experimental.pallas import tpu_sc as plsc

CS = 72
CT = 9
L = 16
NC = 2
NS = 16
NW = NC * NS
SPW = 128
NBUF = 3
PLANE = (27, 8, 128)


def _paint_sample(posv, colv, buf, lb, r, lane, v255):
    prow = posv[lb]
    c_lo = prow[2]
    c_hi = prow[3]
    born = r < prow[1]
    lbv = jnp.full((L,), 0, jnp.int32) + lb
    crow = colv[lb]
    vals = [
        jnp.where(born, jnp.full((L,), 0.0, jnp.float32) + crow[ch], v255)
        for ch in range(3)
    ]
    nk = lax.shift_right_logical(c_hi - c_lo + (L - 1), 4)

    def ck(k2, carry):
        c = c_lo + k2 * L + lane
        msk = c < c_hi
        ct = lax.shift_right_logical(c, 3)
        c8 = c & 7
        for ch in range(3):
            plsc.store_scatter(buf, [ct + 9 * ch, c8, lbv], vals[ch], mask=msk)
        return carry

    lax.fori_loop(0, nk, ck, 0)


def _update_row(posv, colv, rlov, rhiv, buf, r, prevr, lane, v255):
    for k in range(SPW // L):
        rlo = rlov[pl.ds(k * L, L)]
        rhi = rhiv[pl.ds(k * L, L)]
        act_r = (r >= rlo) & (r < rhi)
        if prevr is None:
            ev = act_r
        else:
            born = act_r & (rlo > prevr)
            dead = (prevr >= rlo) & (prevr < rhi) & (rhi <= r)
            ev = born | dead
        cnt = plsc.all_reduce_population_count(ev)[0]

        def body(t, mc):
            la = plsc.all_reduce_ffs(mc)[0]
            _paint_sample(posv, colv, buf, k * L + la, r, lane, v255)
            return mc & (lane != la)

        lax.fori_loop(0, cnt, body, ev)


def _sc_body(pos_hbm, soa_hbm, col_hbm, bg_hbm, out6, posv, colv,
             rlov, rhiv, buf0, buf1, buf2, s0, s1, s2):
    o = out6.reshape(CS * 27, NW, 8, 128)
    wid = lax.axis_index("s") * NC + lax.axis_index("c")
    base = wid * SPW
    nb = 4096

    bufs = [buf0, buf1, buf2]
    sems = [s0, s1, s2]
    lane = lax.iota(jnp.int32, L)
    v255 = jnp.full((L,), 255.0, jnp.float32)

    stages = [
        (pos_hbm.at[pl.ds(base, SPW)], posv),
        (col_hbm.at[pl.ds(base, SPW)], colv),
        (soa_hbm.at[pl.ds(base, SPW)], rlov),
        (soa_hbm.at[pl.ds(nb + base, SPW)], rhiv),
        (bg_hbm, buf0),
        (bg_hbm, buf1),
        (bg_hbm, buf2),
    ]
    for i, (src, dst) in enumerate(stages):
        pltpu.async_copy(src, dst, sems[i % NBUF])
    for i, (src, dst) in enumerate(stages):
        pltpu.make_async_copy(src, dst, sems[i % NBUF]).wait()

    for b in range(NBUF):
        _update_row(posv, colv, rlov, rhiv, bufs[b], b, None, lane, v255)
        pltpu.async_copy(bufs[b], o.at[pl.ds(b * 27, 27), wid], sems[b])

    def group(g, carry):
        for b in range(NBUF):
            r = g * NBUF + b
            prev = r - NBUF
            pltpu.make_async_copy(
                bufs[b], o.at[pl.ds(prev * 27, 27), wid], sems[b]
            ).wait()
            _update_row(posv, colv, rlov, rhiv, bufs[b], r, prev, lane, v255)
            pltpu.async_copy(bufs[b], o.at[pl.ds(r * 27, 27), wid], sems[b])
        return carry

    lax.fori_loop(1, CS // NBUF, group, 0)

    for b in range(NBUF):
        last = CS - NBUF + b
        pltpu.make_async_copy(
            bufs[b], o.at[pl.ds(last * 27, 27), wid], sems[b]
        ).wait()


@jax.jit
def kernel(positions, colors):
    pos = positions.astype(jnp.int32)
    nb = pos.shape[0]
    r_lo = jnp.minimum(pos[:, 0, 0], CS)
    r_hi = jnp.minimum(pos[:, 1, 0], CS)
    c_lo = jnp.minimum(pos[:, 0, 1], CS)
    c_hi = jnp.minimum(pos[:, 1, 1], CS)
    r_hi_eff = jnp.where(c_hi > c_lo, r_hi, 0)

    pos4 = jnp.stack([r_lo, r_hi_eff, c_lo, c_hi], axis=1)
    pos16 = jnp.zeros((nb, L), jnp.int32).at[:, :4].set(pos4)
    soa = jnp.concatenate([r_lo, r_hi_eff])
    col16 = jnp.zeros((nb, L), jnp.float32).at[:, :3].set(colors)
    bg = jnp.full(PLANE, 255.0, jnp.float32)

    assert nb == SPW * NW

    sc_call = pl.kernel(
        _sc_body,
        out_type=jax.ShapeDtypeStruct((CS, 3, CT, NW, 8, 128), jnp.float32),
        mesh=plsc.VectorSubcoreMesh(core_axis_name="c", subcore_axis_name="s"),
        compiler_params=pltpu.CompilerParams(needs_layout_passes=False),
        scratch_types=[
            pltpu.VMEM((SPW, L), jnp.int32),
            pltpu.VMEM((SPW, L), jnp.float32),
            pltpu.VMEM((SPW,), jnp.int32),
            pltpu.VMEM((SPW,), jnp.int32),
            pltpu.VMEM(PLANE, jnp.float32),
            pltpu.VMEM(PLANE, jnp.float32),
            pltpu.VMEM(PLANE, jnp.float32),
            pltpu.SemaphoreType.DMA,
            pltpu.SemaphoreType.DMA,
            pltpu.SemaphoreType.DMA,
        ],
    )
    out6 = sc_call(pos16, soa, col16.reshape(nb, L), bg)
    return jnp.transpose(out6, (3, 5, 0, 2, 4, 1)).reshape(nb, CS, CS, 3)
